# baseline (speedup 1.0000x reference)
import functools

import jax
import jax.numpy as jnp
from jax import lax
from jax.experimental import pallas as pl
from jax.experimental.pallas import tpu as pltpu
from jax.experimental.pallas import tpu_sc as plsc

NC = 2
NS = 16
NW = NC * NS

G = 128
CH = 512


NB = 2
K = CH // G


def _gather_body(ids_hbm, table_hbm, out_hbm, idx_v, rows_v, sems):
    wid = lax.axis_index("s") * NC + lax.axis_index("c")
    rw = idx_v.shape[0] * idx_v.shape[1]
    nch = rw // CH
    base = wid * rw
    pltpu.sync_copy(ids_hbm.at[wid], idx_v)

    def start(c, b):
        for j in range(K):
            pltpu.async_copy(
                table_hbm.at[idx_v.at[c * K + j]],
                rows_v.at[b, pl.ds(j * G, G)],
                sems.at[b],
            )

    def drain_and_write(c, b):
        pltpu.make_async_copy(
            table_hbm.at[pl.ds(0, CH)], rows_v.at[b], sems.at[b]
        ).wait()
        pltpu.sync_copy(rows_v.at[b], out_hbm.at[pl.ds(base + c * CH, CH)])

    for b in range(NB):
        start(b, b)

    def group(g, _):
        for b in range(NB):
            c = g * NB + b
            drain_and_write(c, b)
            start(c + NB, b)
        return 0

    lax.fori_loop(0, nch // NB - 1, group, 0)
    for b in range(NB):
        drain_and_write(nch - NB + b, b)


def _sc_gather(ids, emb_table):
    n = ids.shape[0]
    hid = emb_table.shape[1]
    rw = n // NW
    ids3 = ids.reshape(NW, rw // G, G)
    kern = functools.partial(
        pl.kernel,
        out_type=jax.ShapeDtypeStruct((n, hid), jnp.float32),
        mesh=plsc.VectorSubcoreMesh(core_axis_name="c", subcore_axis_name="s"),
        scratch_types=[
            pltpu.VMEM((rw // G, G), jnp.int32),
            pltpu.VMEM((NB, CH, hid), jnp.float32),
            pltpu.SemaphoreType.DMA((NB,)),
        ],
        compiler_params=pltpu.CompilerParams(use_tc_tiling_on_sc=False),
    )(_gather_body)
    return kern(ids3, emb_table)


def _proj_body(x_ref, m_ref, w_ref, b_ref, o_ref):
    x = x_ref[...]
    y = lax.dot_general(
        x, w_ref[...], (((1,), (1,)), ((), ())),
        preferred_element_type=jnp.float32,
    )
    y = y + b_ref[...]
    o_ref[...] = y * m_ref[...].astype(jnp.float32)


def _tc_project(rows, mask, W, b, blk=4096):
    n, hid = rows.shape
    grid = n // blk
    return pl.pallas_call(
        _proj_body,
        grid=(grid,),
        in_specs=[
            pl.BlockSpec((blk, hid), lambda i: (i, 0)),
            pl.BlockSpec((blk, 1), lambda i: (i, 0)),
            pl.BlockSpec((hid, hid), lambda i: (0, 0)),
            pl.BlockSpec((1, hid), lambda i: (0, 0)),
        ],
        out_specs=pl.BlockSpec((blk, hid), lambda i: (i, 0)),
        out_shape=jax.ShapeDtypeStruct((n, hid), jnp.float32),
    )(rows, mask, W, b)


def kernel(token_ids, attention_mask, emb_table, W, b):
    bsz, seq = token_ids.shape
    hid = emb_table.shape[1]
    n = bsz * seq
    ids = (token_ids * attention_mask).reshape(n)
    rows = _sc_gather(ids, emb_table)
    mask2 = attention_mask.reshape(n, 1)
    out = _tc_project(rows, mask2, W, b.reshape(1, hid))
    return out.reshape(bsz, seq, hid)


# speedup vs baseline: 1.4835x; 1.4835x over previous
import functools

import jax
import jax.numpy as jnp
from jax import lax
from jax.experimental import pallas as pl
from jax.experimental.pallas import tpu as pltpu
from jax.experimental.pallas import tpu_sc as plsc

NC = 2
NS = 16
NW = NC * NS

G = 128
CH = 512
NB = 2
K = CH // G


def _prep_body(e_ref, w_ref, b_ref, o_ref):
    y = lax.dot_general(
        e_ref[...], w_ref[...], (((1,), (1,)), ((), ())),
        preferred_element_type=jnp.float32,
    )
    o_ref[...] = (y + b_ref[...]).astype(jnp.bfloat16)


def _tc_prepass(emb_table, W, b, blk=2048):
    v, hid = emb_table.shape
    return pl.pallas_call(
        _prep_body,
        grid=(v // blk,),
        in_specs=[
            pl.BlockSpec((blk, hid), lambda i: (i, 0)),
            pl.BlockSpec((hid, hid), lambda i: (0, 0)),
            pl.BlockSpec((1, hid), lambda i: (0, 0)),
        ],
        out_specs=pl.BlockSpec((blk, hid), lambda i: (i, 0)),
        out_shape=jax.ShapeDtypeStruct((v, hid), jnp.bfloat16),
    )(emb_table, W, b.reshape(1, hid))


def _gather_body(ids_hbm, table_hbm, out_hbm, idx_v, rows_v, sems):
    wid = lax.axis_index("s") * NC + lax.axis_index("c")
    rw = idx_v.shape[0] * idx_v.shape[1]
    nch = rw // CH
    base = wid * rw
    pltpu.sync_copy(ids_hbm.at[wid], idx_v)

    def start(c, bf):
        for j in range(K):
            pltpu.async_copy(
                table_hbm.at[idx_v.at[c * K + j]],
                rows_v.at[bf, pl.ds(j * G, G)],
                sems.at[bf],
            )

    def drain_and_write(c, bf):
        pltpu.make_async_copy(
            table_hbm.at[pl.ds(0, CH)], rows_v.at[bf], sems.at[bf]
        ).wait()
        pltpu.sync_copy(rows_v.at[bf], out_hbm.at[pl.ds(base + c * CH, CH)])

    for bf in range(NB):
        start(bf, bf)

    def group(g, _):
        for bf in range(NB):
            c = g * NB + bf
            drain_and_write(c, bf)
            start(c + NB, bf)
        return 0

    lax.fori_loop(0, nch // NB - 1, group, 0)
    for bf in range(NB):
        drain_and_write(nch - NB + bf, bf)


def _sc_gather(ids, table_bf16):
    n = ids.shape[0]
    hid = table_bf16.shape[1]
    rw = n // NW
    ids3 = ids.reshape(NW, rw // G, G)
    kern = functools.partial(
        pl.kernel,
        out_type=jax.ShapeDtypeStruct((n, hid), jnp.bfloat16),
        mesh=plsc.VectorSubcoreMesh(core_axis_name="c", subcore_axis_name="s"),
        scratch_types=[
            pltpu.VMEM((rw // G, G), jnp.int32),
            pltpu.VMEM((NB, CH, hid), jnp.bfloat16),
            pltpu.SemaphoreType.DMA((NB,)),
        ],
        compiler_params=pltpu.CompilerParams(use_tc_tiling_on_sc=False),
    )(_gather_body)
    return kern(ids3, table_bf16)


def _fin_body(x_ref, m_ref, o_ref):
    o_ref[...] = x_ref[...].astype(jnp.float32) * m_ref[...].astype(jnp.float32)


def _tc_finish(rows, mask2, blk=4096):
    n, hid = rows.shape
    return pl.pallas_call(
        _fin_body,
        grid=(n // blk,),
        in_specs=[
            pl.BlockSpec((blk, hid), lambda i: (i, 0)),
            pl.BlockSpec((blk, 1), lambda i: (i, 0)),
        ],
        out_specs=pl.BlockSpec((blk, hid), lambda i: (i, 0)),
        out_shape=jax.ShapeDtypeStruct((n, hid), jnp.float32),
    )(rows, mask2)


def kernel(token_ids, attention_mask, emb_table, W, b):
    bsz, seq = token_ids.shape
    hid = emb_table.shape[1]
    n = bsz * seq
    table_bf16 = _tc_prepass(emb_table, W, b)
    ids = (token_ids * attention_mask).reshape(n)
    rows = _sc_gather(ids, table_bf16)
    out = _tc_finish(rows, attention_mask.reshape(n, 1))
    return out.reshape(bsz, seq, hid)
